# baseline (speedup 1.0000x reference)
import functools

import jax
import jax.numpy as jnp
from jax import lax
from jax.experimental import pallas as pl
from jax.experimental.pallas import tpu as pltpu
from jax.experimental.pallas import tpu_sc as plsc

VOCAB = 100000
EMBED_DIM = 64
BATCH = 1024
CTX = 50
CTX_PAD = 56

NUM_CORES = 2
NUM_SUBCORES = 16
NUM_WORKERS = NUM_CORES * NUM_SUBCORES
BPW = BATCH // NUM_WORKERS
LANES = 16
DVECS = EMBED_DIM // LANES
IDX_PW = BPW * CTX_PAD
CHUNK = 128
NCHUNKS = IDX_PW // CHUNK
TBL_W = EMBED_DIM

_sc_mesh = plsc.VectorSubcoreMesh(
    core_axis_name="c", subcore_axis_name="s",
    num_cores=NUM_CORES, num_subcores=NUM_SUBCORES)


@functools.partial(
    pl.kernel,
    out_type=jax.ShapeDtypeStruct((BATCH, EMBED_DIM), jnp.float32),
    mesh=_sc_mesh,
    scratch_types=[
        pltpu.VMEM((IDX_PW,), jnp.int32),
        pltpu.VMEM((IDX_PW // 4, TBL_W), jnp.float32),
        pltpu.VMEM((IDX_PW // 4, TBL_W), jnp.float32),
        pltpu.VMEM((BPW, EMBED_DIM), jnp.float32),
        pltpu.SemaphoreType.DMA,
        pltpu.SemaphoreType.DMA,
    ],
    compiler_params=pltpu.CompilerParams(use_tc_tiling_on_sc=False),
)
def _pool_sc(ctx_hbm, table_hbm, out_hbm, idx_v, rows0, rows1, pooled_v,
             sem0, sem1):
    wid = lax.axis_index("s") * NUM_CORES + lax.axis_index("c")
    pltpu.sync_copy(ctx_hbm.at[pl.ds(wid * IDX_PW, IDX_PW)], idx_v)

    NPH = 4
    SPP = IDX_PW // NPH // LANES
    RPP = BPW // NPH
    rows = (rows0, rows1)
    sems = (sem0, sem1)
    inv = jnp.float32(1.0 / CTX)

    def stream(p, j):
        g = p * SPP + j
        vec = idx_v[pl.ds(g * LANES, LANES)]
        return pltpu.make_async_copy(
            table_hbm.at[vec],
            rows[p % 2].at[pl.ds(j * LANES, LANES)], sems[p % 2])

    def fire(p):
        for j in range(SPP):
            stream(p, j).start()

    def drain(p):
        for j in range(SPP):
            stream(p, j).wait()

    def pool(p):
        buf = rows[p % 2]

        def pool_row(r, carry):
            base = r * CTX_PAD
            for d in range(DVECS):
                acc = buf[base, pl.ds(d * LANES, LANES)]
                for c in range(1, CTX):
                    acc = acc + buf[base + c, pl.ds(d * LANES, LANES)]
                pooled_v[p * RPP + r, pl.ds(d * LANES, LANES)] = acc * inv
            return carry

        lax.fori_loop(0, RPP, pool_row, 0)

    fire(0)
    fire(1)
    drain(0)
    pool(0)
    fire(2)
    drain(1)
    pool(1)
    fire(3)
    drain(2)
    pool(2)
    drain(3)
    pool(3)
    pltpu.sync_copy(pooled_v, out_hbm.at[pl.ds(wid * BPW, BPW)])


VTILE = 2048
NBUF = 4
NFULL = VOCAB // VTILE
VTAIL = VOCAB - NFULL * VTILE


def _dot_bias(p_v, w_ref, b_ref):
    prod = lax.dot_general(
        w_ref[...], p_v[...],
        dimension_numbers=(((1,), (0,)), ((), ())),
        preferred_element_type=jnp.float32,
    )
    return prod + b_ref[...]


def _proj_kernel(p_hbm, w_hbm, b_hbm, o_hbm, p_v, b_tile, b_tail, w_bufs,
                 acc_bufs, w_tail, acc_tail, sems, sem_tail):
    pltpu.sync_copy(p_hbm, p_v)

    def out_dma(k, t):
        return pltpu.make_async_copy(
            acc_bufs.at[k], o_hbm.at[pl.ds(t * VTILE, VTILE)], sems.at[k])

    def step(i, carry):
        for k in range(NBUF):
            t = i * NBUF + k
            pltpu.sync_copy(w_hbm.at[pl.ds(t * VTILE, VTILE)], w_bufs.at[k])
            pltpu.sync_copy(b_hbm.at[pl.ds(t * VTILE, VTILE)], b_tile)

            @pl.when(i > 0)
            def _():
                out_dma(k, t - NBUF).wait()

            acc_bufs[k] = _dot_bias(p_v, w_bufs.at[k], b_tile)
            out_dma(k, t).start()
        return carry

    lax.fori_loop(0, NFULL // NBUF, step, 0, unroll=False)

    pltpu.sync_copy(w_hbm.at[pl.ds(NFULL * VTILE, VTAIL)], w_tail)
    pltpu.sync_copy(b_hbm.at[pl.ds(NFULL * VTILE, VTAIL)], b_tail)
    acc_tail[...] = _dot_bias(p_v, w_tail, b_tail)
    pltpu.make_async_copy(
        acc_tail, o_hbm.at[pl.ds(NFULL * VTILE, VTAIL)], sem_tail).start()

    for k in range(NBUF):
        out_dma(k, NFULL - NBUF + k).wait()
    pltpu.make_async_copy(
        acc_tail, o_hbm.at[pl.ds(NFULL * VTILE, VTAIL)], sem_tail).wait()


def _project(pooled, lin_w, lin_b2d):
    return pl.pallas_call(
        _proj_kernel,
        in_specs=[
            pl.BlockSpec(memory_space=pl.ANY),
            pl.BlockSpec(memory_space=pl.ANY),
            pl.BlockSpec(memory_space=pl.ANY),
        ],
        out_specs=pl.BlockSpec(memory_space=pl.ANY),
        out_shape=jax.ShapeDtypeStruct((VOCAB, BATCH), jnp.float32),
        scratch_shapes=[
            pltpu.VMEM((EMBED_DIM, BATCH), jnp.bfloat16),
            pltpu.VMEM((VTILE, 1), jnp.float32),
            pltpu.VMEM((VTAIL, 1), jnp.float32),
            pltpu.VMEM((NBUF, VTILE, EMBED_DIM), jnp.bfloat16),
            pltpu.VMEM((NBUF, VTILE, BATCH), jnp.float32),
            pltpu.VMEM((VTAIL, EMBED_DIM), jnp.bfloat16),
            pltpu.VMEM((VTAIL, BATCH), jnp.float32),
            pltpu.SemaphoreType.DMA((NBUF,)),
            pltpu.SemaphoreType.DMA,
        ],
    )(pooled, lin_w, lin_b2d)


def kernel(context, emb_table, lin_w, lin_b):
    ctx = context.astype(jnp.int32)
    ctx_pad = jnp.pad(ctx, ((0, 0), (0, CTX_PAD - CTX)))
    ctx_flat = ctx_pad.reshape(BATCH * CTX_PAD)
    pooled = _pool_sc(ctx_flat, emb_table)
    out_t = _project(pooled.T.astype(jnp.bfloat16),
                     lin_w.astype(jnp.bfloat16),
                     lin_b.reshape(VOCAB, 1))
    return out_t.T


# speedup vs baseline: 1.2488x; 1.2488x over previous
import functools

import jax
import jax.numpy as jnp
from jax import lax
from jax.experimental import pallas as pl
from jax.experimental.pallas import tpu as pltpu
from jax.experimental.pallas import tpu_sc as plsc

VOCAB = 100000
EMBED_DIM = 64
BATCH = 1024
CTX = 50
CTX_PAD = 56

NUM_CORES = 2
NUM_SUBCORES = 16
NUM_WORKERS = NUM_CORES * NUM_SUBCORES
BPW = BATCH // NUM_WORKERS
LANES = 16
DVECS = EMBED_DIM // LANES
IDX_PW = BPW * CTX_PAD
CHUNK = 128
NCHUNKS = IDX_PW // CHUNK
TBL_W = EMBED_DIM

_sc_mesh = plsc.VectorSubcoreMesh(
    core_axis_name="c", subcore_axis_name="s",
    num_cores=NUM_CORES, num_subcores=NUM_SUBCORES)


@functools.partial(
    pl.kernel,
    out_type=jax.ShapeDtypeStruct((BATCH, EMBED_DIM), jnp.float32),
    mesh=_sc_mesh,
    scratch_types=[
        pltpu.VMEM((IDX_PW,), jnp.int32),
        pltpu.VMEM((IDX_PW // 4, TBL_W), jnp.float32),
        pltpu.VMEM((IDX_PW // 4, TBL_W), jnp.float32),
        pltpu.VMEM((BPW, EMBED_DIM), jnp.float32),
        pltpu.SemaphoreType.DMA,
        pltpu.SemaphoreType.DMA,
    ],
    compiler_params=pltpu.CompilerParams(use_tc_tiling_on_sc=False),
)
def _pool_sc(ctx_hbm, table_hbm, out_hbm, idx_v, rows0, rows1, pooled_v,
             sem0, sem1):
    wid = lax.axis_index("s") * NUM_CORES + lax.axis_index("c")
    pltpu.sync_copy(ctx_hbm.at[pl.ds(wid * IDX_PW, IDX_PW)], idx_v)

    NPH = 4
    SPP = IDX_PW // NPH // LANES
    RPP = BPW // NPH
    rows = (rows0, rows1)
    sems = (sem0, sem1)
    inv = jnp.float32(1.0 / CTX)

    def stream(p, j):
        g = p * SPP + j
        vec = idx_v[pl.ds(g * LANES, LANES)]
        return pltpu.make_async_copy(
            table_hbm.at[vec],
            rows[p % 2].at[pl.ds(j * LANES, LANES)], sems[p % 2])

    def fire(p):
        for j in range(SPP):
            stream(p, j).start()

    def drain(p):
        for j in range(SPP):
            stream(p, j).wait()

    def pool(p):
        buf = rows[p % 2]

        def pool_row(r, carry):
            base = r * CTX_PAD
            for d in range(DVECS):
                acc = buf[base, pl.ds(d * LANES, LANES)]
                for c in range(1, CTX):
                    acc = acc + buf[base + c, pl.ds(d * LANES, LANES)]
                pooled_v[p * RPP + r, pl.ds(d * LANES, LANES)] = acc * inv
            return carry

        lax.fori_loop(0, RPP, pool_row, 0)

    fire(0)
    fire(1)
    drain(0)
    pool(0)
    fire(2)
    drain(1)
    pool(1)
    fire(3)
    drain(2)
    pool(2)
    drain(3)
    pool(3)
    pltpu.sync_copy(pooled_v, out_hbm.at[pl.ds(wid * BPW, BPW)])


VTILE = 2048
NBUF = 4
NFULL = VOCAB // VTILE
VTAIL = VOCAB - NFULL * VTILE


def _dot(p_v, w_ref):
    return lax.dot_general(
        w_ref[...], p_v[...],
        dimension_numbers=(((1,), (0,)), ((), ())),
        preferred_element_type=jnp.float32,
    )


def _proj_kernel(p_hbm, w_hbm, o_hbm, p_v, w_bufs, acc_bufs,
                 w_tail, acc_tail, sems, sem_tail):
    pltpu.sync_copy(p_hbm, p_v)

    def out_dma(k, t):
        return pltpu.make_async_copy(
            acc_bufs.at[k], o_hbm.at[pl.ds(t * VTILE, VTILE)], sems.at[k])

    def step(i, carry):
        for k in range(NBUF):
            t = i * NBUF + k
            pltpu.sync_copy(w_hbm.at[pl.ds(t * VTILE, VTILE)], w_bufs.at[k])

            @pl.when(i > 0)
            def _():
                out_dma(k, t - NBUF).wait()

            acc_bufs[k] = _dot(p_v, w_bufs.at[k])
            out_dma(k, t).start()
        return carry

    lax.fori_loop(0, NFULL // NBUF, step, 0, unroll=False)

    pltpu.sync_copy(w_hbm.at[pl.ds(NFULL * VTILE, VTAIL)], w_tail)
    acc_tail[...] = _dot(p_v, w_tail)
    pltpu.make_async_copy(
        acc_tail, o_hbm.at[pl.ds(NFULL * VTILE, VTAIL)], sem_tail).start()

    for k in range(NBUF):
        out_dma(k, NFULL - NBUF + k).wait()
    pltpu.make_async_copy(
        acc_tail, o_hbm.at[pl.ds(NFULL * VTILE, VTAIL)], sem_tail).wait()


KAUG = EMBED_DIM + 1


def _project(pooled_t_aug, w_aug):
    return pl.pallas_call(
        _proj_kernel,
        in_specs=[
            pl.BlockSpec(memory_space=pl.ANY),
            pl.BlockSpec(memory_space=pl.ANY),
        ],
        out_specs=pl.BlockSpec(memory_space=pl.ANY),
        out_shape=jax.ShapeDtypeStruct((VOCAB, BATCH), jnp.float32),
        scratch_shapes=[
            pltpu.VMEM((KAUG, BATCH), jnp.bfloat16),
            pltpu.VMEM((NBUF, VTILE, KAUG), jnp.bfloat16),
            pltpu.VMEM((NBUF, VTILE, BATCH), jnp.float32),
            pltpu.VMEM((VTAIL, KAUG), jnp.bfloat16),
            pltpu.VMEM((VTAIL, BATCH), jnp.float32),
            pltpu.SemaphoreType.DMA((NBUF,)),
            pltpu.SemaphoreType.DMA,
        ],
    )(pooled_t_aug, w_aug)


def kernel(context, emb_table, lin_w, lin_b):
    ctx = context.astype(jnp.int32)
    ctx_pad = jnp.pad(ctx, ((0, 0), (0, CTX_PAD - CTX)))
    ctx_flat = ctx_pad.reshape(BATCH * CTX_PAD)
    pooled = _pool_sc(ctx_flat, emb_table)
    p_aug = jnp.concatenate(
        [pooled.T, jnp.ones((1, BATCH), jnp.float32)], axis=0)
    w_aug = jnp.concatenate([lin_w, lin_b.reshape(VOCAB, 1)], axis=1)
    out_t = _project(p_aug.astype(jnp.bfloat16), w_aug.astype(jnp.bfloat16))
    return out_t.T
